# Initial kernel scaffold; baseline (speedup 1.0000x reference)
import functools

import jax
import jax.numpy as jnp
from jax import lax
from jax.experimental import pallas as pl
from jax.experimental.pallas import tpu as pltpu
from jax.experimental.pallas import tpu_sc as plsc

B = 16384
NS = 26
ND = 13
D = 32
NF = NS + ND

NC = 2
NSUB = 16
NW = NC * NSUB

SP_TOT = B * NS
DN_TOT = B * ND
SP_W = SP_TOT // NW
DN_W = DN_TOT // NW
CH = 1664
SP_CHUNKS = SP_W // CH
DN_CHUNKS = DN_W // CH


def _matmul_body(x_ref, w_ref, o_ref):
    o_ref[...] = jnp.dot(x_ref[...], w_ref[...],
                         preferred_element_type=jnp.float32)


def _dense_proj(dense_inputs, w_dense):
    bm = 2048
    return pl.pallas_call(
        _matmul_body,
        grid=(B // bm,),
        in_specs=[
            pl.BlockSpec((bm, ND), lambda i: (i, 0)),
            pl.BlockSpec((ND, ND * D), lambda i: (0, 0)),
        ],
        out_specs=pl.BlockSpec((bm, ND * D), lambda i: (i, 0)),
        out_shape=jax.ShapeDtypeStruct((B, ND * D), jnp.float32),
    )(dense_inputs, w_dense)


_mesh = plsc.VectorSubcoreMesh(core_axis_name="c", subcore_axis_name="s")


@functools.partial(
    pl.kernel,
    out_type=jax.ShapeDtypeStruct((B * NF, D), jnp.float32),
    mesh=_mesh,
    scratch_types=[
        pltpu.VMEM((CH,), jnp.int32),
        pltpu.VMEM((CH,), jnp.int32),
        pltpu.VMEM((CH, D), jnp.float32),
        pltpu.SemaphoreType.DMA,
        pltpu.SemaphoreType.DMA,
    ],
)
def _sc_embed(idx_hbm, dsp_hbm, drows_hbm, dsd_hbm, table_hbm, out_hbm,
              idx_v, didx_v, rows_v, gsem, ssem):
    wid = lax.axis_index("s") * NC + lax.axis_index("c")

    def sp_body(c, _):
        base = wid * SP_W + c * CH
        pltpu.sync_copy(idx_hbm.at[pl.ds(base, CH)], idx_v)
        pltpu.async_copy(table_hbm.at[idx_v], rows_v, gsem).wait()
        pltpu.sync_copy(dsp_hbm.at[pl.ds(base, CH)], didx_v)
        pltpu.async_copy(rows_v, out_hbm.at[didx_v], ssem).wait()
        return 0

    lax.fori_loop(0, SP_CHUNKS, sp_body, 0)

    def dn_body(c, _):
        base = wid * DN_W + c * CH
        pltpu.sync_copy(drows_hbm.at[pl.ds(base, CH)], rows_v)
        pltpu.sync_copy(dsd_hbm.at[pl.ds(base, CH)], didx_v)
        pltpu.async_copy(rows_v, out_hbm.at[didx_v], ssem).wait()
        return 0

    lax.fori_loop(0, DN_CHUNKS, dn_body, 0)


def kernel(sparse_inputs, dense_inputs, emb_table, W_dense):
    dense_rows = _dense_proj(dense_inputs, W_dense).reshape(DN_TOT, D)
    flat_idx = sparse_inputs.astype(jnp.int32).reshape(SP_TOT)
    brow = jnp.arange(B, dtype=jnp.int32)[:, None] * NF
    dsp = (brow + jnp.arange(NS, dtype=jnp.int32)[None, :]).reshape(SP_TOT)
    dsd = (brow + NS + jnp.arange(ND, dtype=jnp.int32)[None, :]).reshape(DN_TOT)
    out = _sc_embed(flat_idx, dsp, dense_rows, dsd, emb_table)
    return out.reshape(B, NF, D)


# speedup vs baseline: 1.3792x; 1.3792x over previous
import functools

import jax
import jax.numpy as jnp
from jax import lax
from jax.experimental import pallas as pl
from jax.experimental.pallas import tpu as pltpu
from jax.experimental.pallas import tpu_sc as plsc

B = 16384
NS = 26
ND = 13
D = 32
NF = NS + ND

NC = 2
NSUB = 16
NW = NC * NSUB

SP_TOT = B * NS
DN_TOT = B * ND
SP_W = SP_TOT // NW
DN_W = DN_TOT // NW
CH = 1664
SP_CHUNKS = SP_W // CH
DN_CHUNKS = DN_W // CH


def _matmul_body(x_ref, w_ref, o_ref):
    o_ref[...] = jnp.dot(x_ref[...], w_ref[...],
                         preferred_element_type=jnp.float32)


def _dense_proj(dense_inputs, w_dense):
    bm = 2048
    return pl.pallas_call(
        _matmul_body,
        grid=(B // bm,),
        in_specs=[
            pl.BlockSpec((bm, ND), lambda i: (i, 0)),
            pl.BlockSpec((ND, ND * D), lambda i: (0, 0)),
        ],
        out_specs=pl.BlockSpec((bm, ND * D), lambda i: (i, 0)),
        out_shape=jax.ShapeDtypeStruct((B, ND * D), jnp.float32),
    )(dense_inputs, w_dense)


_mesh = plsc.VectorSubcoreMesh(core_axis_name="c", subcore_axis_name="s")


@functools.partial(
    pl.kernel,
    out_type=jax.ShapeDtypeStruct((B * NF, D), jnp.float32),
    mesh=_mesh,
    scratch_types=[
        pltpu.VMEM((CH,), jnp.int32),
        pltpu.VMEM((CH,), jnp.int32),
        pltpu.VMEM((CH, D), jnp.float32),
        pltpu.SemaphoreType.DMA,
        pltpu.SemaphoreType.DMA,
    ],
    compiler_params=pltpu.CompilerParams(use_tc_tiling_on_sc=False),
)
def _sc_embed(idx_hbm, dsp_hbm, drows_hbm, dsd_hbm, table_hbm, out_hbm,
              idx_v, didx_v, rows_v, gsem, ssem):
    wid = lax.axis_index("s") * NC + lax.axis_index("c")

    def sp_body(c, _):
        base = wid * SP_W + c * CH
        pltpu.sync_copy(idx_hbm.at[pl.ds(base, CH)], idx_v)
        pltpu.async_copy(table_hbm.at[idx_v], rows_v, gsem).wait()
        pltpu.sync_copy(dsp_hbm.at[pl.ds(base, CH)], didx_v)
        pltpu.async_copy(rows_v, out_hbm.at[didx_v], ssem).wait()
        return 0

    lax.fori_loop(0, SP_CHUNKS, sp_body, 0)

    def dn_body(c, _):
        base = wid * DN_W + c * CH
        pltpu.sync_copy(drows_hbm.at[pl.ds(base, CH)], rows_v)
        pltpu.sync_copy(dsd_hbm.at[pl.ds(base, CH)], didx_v)
        pltpu.async_copy(rows_v, out_hbm.at[didx_v], ssem).wait()
        return 0

    lax.fori_loop(0, DN_CHUNKS, dn_body, 0)


def kernel(sparse_inputs, dense_inputs, emb_table, W_dense):
    dense_rows = _dense_proj(dense_inputs, W_dense).reshape(DN_TOT, D)
    flat_idx = sparse_inputs.astype(jnp.int32).reshape(SP_TOT)
    brow = jnp.arange(B, dtype=jnp.int32)[:, None] * NF
    dsp = (brow + jnp.arange(NS, dtype=jnp.int32)[None, :]).reshape(SP_TOT)
    dsd = (brow + NS + jnp.arange(ND, dtype=jnp.int32)[None, :]).reshape(DN_TOT)
    out = _sc_embed(flat_idx, dsp, dense_rows, dsd, emb_table)
    return out.reshape(B, NF, D)


# speedup vs baseline: 1.4055x; 1.0191x over previous
import functools

import jax
import jax.numpy as jnp
from jax import lax
from jax.experimental import pallas as pl
from jax.experimental.pallas import tpu as pltpu
from jax.experimental.pallas import tpu_sc as plsc

B = 16384
NS = 26
ND = 13
D = 32
NF = NS + ND

NC = 2
NSUB = 16
NW = NC * NSUB

SP_TOT = B * NS
DN_TOT = B * ND
SP_W = SP_TOT // NW
DN_W = DN_TOT // NW
CH = 832
SP_CHUNKS = SP_W // CH
DN_CHUNKS = DN_W // CH
N_CHUNKS = SP_CHUNKS + DN_CHUNKS
NB = 3


def _matmul_body(x_ref, w_ref, o_ref):
    o_ref[...] = jnp.dot(x_ref[...], w_ref[...],
                         preferred_element_type=jnp.float32)


def _dense_proj(dense_inputs, w_dense):
    bm = 2048
    return pl.pallas_call(
        _matmul_body,
        grid=(B // bm,),
        in_specs=[
            pl.BlockSpec((bm, ND), lambda i: (i, 0)),
            pl.BlockSpec((ND, ND * D), lambda i: (0, 0)),
        ],
        out_specs=pl.BlockSpec((bm, ND * D), lambda i: (i, 0)),
        out_shape=jax.ShapeDtypeStruct((B, ND * D), jnp.float32),
    )(dense_inputs, w_dense)


_mesh = plsc.VectorSubcoreMesh(core_axis_name="c", subcore_axis_name="s")


@functools.partial(
    pl.kernel,
    out_type=jax.ShapeDtypeStruct((B * NF, D), jnp.float32),
    mesh=_mesh,
    scratch_types=(
        [pltpu.VMEM((SP_W,), jnp.int32)]
        + [pltpu.VMEM((CH,), jnp.int32) for _ in range(NB)]
        + [pltpu.VMEM((CH, D), jnp.float32) for _ in range(NB)]
        + [pltpu.SemaphoreType.DMA for _ in range(1 + 3 * NB)]
    ),
    compiler_params=pltpu.CompilerParams(use_tc_tiling_on_sc=False),
)
def _sc_embed(idx_hbm, dsp_hbm, drows_hbm, dsd_hbm, table_hbm, out_hbm,
              idx_all, *rest):
    dbufs = rest[0:NB]
    rows = rest[NB:2 * NB]
    isem = rest[2 * NB]
    dsems = rest[2 * NB + 1:3 * NB + 1]
    gsems = rest[3 * NB + 1:4 * NB + 1]
    ssems = rest[4 * NB + 1:5 * NB + 1]

    wid = lax.axis_index("s") * NC + lax.axis_index("c")
    sp_base = wid * SP_W
    dn_base = wid * DN_W

    def didx_copy(c):
        b = c % NB
        if c < SP_CHUNKS:
            src = dsp_hbm.at[pl.ds(sp_base + c * CH, CH)]
        else:
            src = dsd_hbm.at[pl.ds(dn_base + (c - SP_CHUNKS) * CH, CH)]
        return pltpu.async_copy(src, dbufs[b], dsems[b])

    def rows_copy(c):
        b = c % NB
        if c < SP_CHUNKS:
            src = table_hbm.at[idx_all.at[pl.ds(c * CH, CH)]]
        else:
            src = drows_hbm.at[pl.ds(dn_base + (c - SP_CHUNKS) * CH, CH)]
        return pltpu.async_copy(src, rows[b], gsems[b])

    ia = pltpu.async_copy(idx_hbm.at[pl.ds(sp_base, SP_W)], idx_all, isem)
    dl = [None] * N_CHUNKS
    gd = [None] * N_CHUNKS
    sd = [None] * N_CHUNKS
    for c in range(NB):
        dl[c] = didx_copy(c)
    ia.wait()
    for c in range(N_CHUNKS):
        if c >= NB:
            sd[c - NB].wait()
            dl[c] = didx_copy(c)
        gd[c] = rows_copy(c)
        if c >= 1:
            pb = (c - 1) % NB
            gd[c - 1].wait()
            dl[c - 1].wait()
            sd[c - 1] = pltpu.async_copy(rows[pb], out_hbm.at[dbufs[pb]],
                                         ssems[pb])
    lb = (N_CHUNKS - 1) % NB
    gd[N_CHUNKS - 1].wait()
    dl[N_CHUNKS - 1].wait()
    sd[N_CHUNKS - 1] = pltpu.async_copy(
        rows[lb], out_hbm.at[dbufs[lb]], ssems[lb])
    for c in range(N_CHUNKS - NB, N_CHUNKS):
        sd[c].wait()


def kernel(sparse_inputs, dense_inputs, emb_table, W_dense):
    dense_rows = _dense_proj(dense_inputs, W_dense).reshape(DN_TOT, D)
    flat_idx = sparse_inputs.astype(jnp.int32).reshape(SP_TOT)
    brow = jnp.arange(B, dtype=jnp.int32)[:, None] * NF
    dsp = (brow + jnp.arange(NS, dtype=jnp.int32)[None, :]).reshape(SP_TOT)
    dsd = (brow + NS + jnp.arange(ND, dtype=jnp.int32)[None, :]).reshape(DN_TOT)
    out = _sc_embed(flat_idx, dsp, dense_rows, dsd, emb_table)
    return out.reshape(B, NF, D)
